# baseline (speedup 1.0000x reference)
import functools

import jax
import jax.numpy as jnp
from jax import lax
from jax.experimental import pallas as pl
from jax.experimental.pallas import tpu as pltpu
from jax.experimental.pallas import tpu_sc as plsc

N = 10000
D = 128
DH = D // 2
E = 320000

NC = 2
NS = 16
L = 16

NP = 10112
SL = NP // NS
EP = 327680
G = 128
CH0 = 64
CH1 = 96
C0TOT = NS * CH0
CHMAX = max(CH0, CH1)
EPTMAX = CHMAX * G


def _proj_body(feats_ref, w_ref, a_ref, h0_ref, h1_ref, elr_ref):
    x = feats_ref[...]
    x = jnp.where(x >= 0.0, x, 0.2 * x)
    h = jnp.dot(x, w_ref[...], preferred_element_type=jnp.float32)
    h0_ref[...] = h[:, :DH]
    h1_ref[...] = h[:, DH:]
    elr_ref[...] = jnp.dot(h, a_ref[...], preferred_element_type=jnp.float32)


def _project(feats, W, A):
    return pl.pallas_call(
        _proj_body,
        out_shape=(
            jax.ShapeDtypeStruct((N, DH), jnp.float32),
            jax.ShapeDtypeStruct((N, DH), jnp.float32),
            jax.ShapeDtypeStruct((N, 2), jnp.float32),
        ),
    )(feats, W, A)


NB = 2


def _edge_body(h0_hbm, h1_hbm, elp_hbm, erp_hbm, srcp_hbm, dstp_hbm,
               num_out, den_out,
               el_v, er_v, eefull, sidx_all, didx_all, rows4, den4,
               num_acc, den_acc, gsem, ssem, dsem):
    cid = lax.axis_index("c")
    sid = lax.axis_index("s")
    base = sid * SL
    nch = jnp.where(cid == 0, CH0, CH1)
    ngrp = nch // NB
    startrow = jnp.where(cid == 0, sid * CH0, C0TOT + sid * CH1)

    pltpu.sync_copy(elp_hbm, el_v)
    pltpu.sync_copy(erp_hbm, er_v)
    pltpu.sync_copy(srcp_hbm.at[pl.ds(startrow, CHMAX)], sidx_all)
    pltpu.sync_copy(dstp_hbm.at[pl.ds(startrow, CHMAX)], didx_all)

    zero = jnp.zeros((L,), jnp.float32)

    def eevec(c, _):
        for j in range(G // L):
            sv = sidx_all[c, pl.ds(j * L, L)]
            dv = didx_all[c, pl.ds(j * L, L)]
            s = plsc.load_gather(el_v, [sv]) + plsc.load_gather(er_v, [dv])
            e = jnp.where(s >= 0.0, s, 0.2 * s)
            eefull[pl.ds(c * G + j * L, L)] = jnp.exp(e)
        return 0

    lax.fori_loop(0, nch, eevec, 0)

    for ph, hp_hbm in ((0, h0_hbm), (1, h1_hbm)):
        def zrow(i, _):
            for v in range(DH // L):
                rows4[0, i, pl.ds(v * L, L)] = zero
            den4[0, i, :] = zero
            return 0

        lax.fori_loop(0, G, zrow, 0)

        for k in range(4):
            pltpu.sync_copy(rows4.at[0], num_acc.at[pl.ds(base + k * G, G)])
            if ph == 0:
                pltpu.sync_copy(den4.at[0], den_acc.at[pl.ds(base + k * G, G)])
        rem = SL - 4 * G
        pltpu.sync_copy(rows4.at[0, pl.ds(0, rem)],
                        num_acc.at[pl.ds(base + 4 * G, rem)])
        if ph == 0:
            pltpu.sync_copy(den4.at[0, pl.ds(0, rem)],
                            den_acc.at[pl.ds(base + 4 * G, rem)])
        plsc.subcore_barrier()

        def group(g, _):
            c0 = g * NB
            gd = [pltpu.async_copy(hp_hbm.at[sidx_all.at[c0 + b]],
                                   rows4.at[b], gsem.at[b])
                  for b in range(NB)]
            outd = []
            for b in range(NB):
                gd[b].wait()
                loc = (c0 + b) * G

                def row4(i, _, b=b, loc=loc):
                    for k in range(4):
                        r = i * 4 + k
                        sp = plsc.load_gather(
                            eefull, [jnp.full((L,), loc, jnp.int32) + r])
                        if ph == 0:
                            den4[b, r, :] = sp
                        for v in range(DH // L):
                            rows4[b, r, pl.ds(v * L, L)] = (
                                rows4[b, r, pl.ds(v * L, L)] * sp)
                    return 0

                lax.fori_loop(0, G // 4, row4, 0)

                outd.append(pltpu.async_copy(
                    rows4.at[b], num_acc.at[didx_all.at[c0 + b]],
                    ssem.at[b], add=True))
                if ph == 0:
                    outd.append(pltpu.async_copy(
                        den4.at[b], den_acc.at[didx_all.at[c0 + b]],
                        dsem.at[b], add=True))
            for d in outd:
                d.wait()
            return 0

        lax.fori_loop(0, ngrp, group, 0)
        plsc.subcore_barrier()

        for k in range(4):
            pltpu.sync_copy(num_acc.at[pl.ds(base + k * G, G)],
                            num_out.at[cid, ph, pl.ds(base + k * G, G)])
            if ph == 0:
                pltpu.sync_copy(den_acc.at[pl.ds(base + k * G, G)],
                                den_out.at[cid, pl.ds(base + k * G, G)])
        pltpu.sync_copy(num_acc.at[pl.ds(base + 4 * G, rem)],
                        num_out.at[cid, ph, pl.ds(base + 4 * G, rem)])
        if ph == 0:
            pltpu.sync_copy(den_acc.at[pl.ds(base + 4 * G, rem)],
                            den_out.at[cid, pl.ds(base + 4 * G, rem)])
        if ph == 0:
            plsc.subcore_barrier()


_edge_kernel = functools.partial(
    pl.kernel,
    out_type=(
        jax.ShapeDtypeStruct((NC, 2, NP, DH), jnp.float32),
        jax.ShapeDtypeStruct((NC, NP, L), jnp.float32),
    ),
    mesh=plsc.VectorSubcoreMesh(core_axis_name="c", subcore_axis_name="s"),
    compiler_params=pltpu.CompilerParams(needs_layout_passes=False,
                                         use_tc_tiling_on_sc=False),
    scratch_types=[
        pltpu.VMEM((NP,), jnp.float32),
        pltpu.VMEM((NP,), jnp.float32),
        pltpu.VMEM((EPTMAX,), jnp.float32),
        pltpu.VMEM((CHMAX, G), jnp.int32),
        pltpu.VMEM((CHMAX, G), jnp.int32),
        pltpu.VMEM((NB, G, DH), jnp.float32),
        pltpu.VMEM((NB, G, L), jnp.float32),
        pltpu.VMEM_SHARED((NP, DH), jnp.float32),
        pltpu.VMEM_SHARED((NP, L), jnp.float32),
        pltpu.SemaphoreType.DMA((NB,)),
        pltpu.SemaphoreType.DMA((NB,)),
        pltpu.SemaphoreType.DMA((NB,)),
    ],
)(_edge_body)


def _final_body(num_ref, den_ref, bias_ref, out_ref):
    d = den_ref[0, :N, :] + den_ref[1, :N, :]
    d0 = d[:, 0:1]
    for ph in range(2):
        n = num_ref[0, ph, :N, :] + num_ref[1, ph, :N, :]
        out_ref[:, ph * DH:(ph + 1) * DH] = (
            jnp.where(d0 > 0.0, n / d0, 0.0)
            + bias_ref[:, ph * DH:(ph + 1) * DH])


def _finalize(num, den, bias2):
    return pl.pallas_call(
        _final_body,
        out_shape=jax.ShapeDtypeStruct((N, D), jnp.float32),
    )(num, den, bias2)


def kernel(feats, edge_index, W, attn_l, attn_r, bias):
    src = edge_index[0].astype(jnp.int32)
    dst = edge_index[1].astype(jnp.int32)
    srcp = jnp.concatenate(
        [src, jnp.zeros((EP - E,), jnp.int32)]).reshape(EP // G, G)
    pad_dst = N + jnp.arange(EP - E, dtype=jnp.int32) % (NP - N)
    dstp = jnp.concatenate([dst, pad_dst]).reshape(EP // G, G)
    A = jnp.stack([attn_l, attn_r], axis=1)

    h0, h1, elr = _project(feats, W, A)
    elp = jnp.concatenate([elr[:, 0], jnp.zeros((NP - N,), jnp.float32)])
    erp = jnp.concatenate([elr[:, 1], jnp.zeros((NP - N,), jnp.float32)])

    num, den = _edge_kernel(h0, h1, elp, erp, srcp, dstp)
    return _finalize(num, den, bias.reshape(1, D))


# speedup vs baseline: 1.2034x; 1.2034x over previous
import functools

import jax
import jax.numpy as jnp
from jax import lax
from jax.experimental import pallas as pl
from jax.experimental.pallas import tpu as pltpu
from jax.experimental.pallas import tpu_sc as plsc

N = 10000
D = 128
DH = D // 2
E = 320000

NC = 2
NS = 16
L = 16

NP = 10112
SL = NP // NS
EP = 327680
G = 128
CH0 = 104
CH1 = 56
C0TOT = NS * CH0
SEG = 56
EPG = EP // G


def _proj_body(feats_ref, w_ref, a_ref, h0_ref, h1_ref, elr_ref):
    x = feats_ref[...]
    x = jnp.where(x >= 0.0, x, 0.2 * x)
    h = jnp.dot(x, w_ref[...], preferred_element_type=jnp.float32)
    h0_ref[...] = h[:, :DH]
    h1_ref[...] = h[:, DH:]
    elr_ref[...] = jnp.dot(h, a_ref[...], preferred_element_type=jnp.float32)


def _project(feats, W, A):
    return pl.pallas_call(
        _proj_body,
        out_shape=(
            jax.ShapeDtypeStruct((N, DH), jnp.float32),
            jax.ShapeDtypeStruct((N, DH), jnp.float32),
            jax.ShapeDtypeStruct((N, 2), jnp.float32),
        ),
    )(feats, W, A)


NB = 2


def _edge_body(h0_hbm, h1_hbm, elp_hbm, erp_hbm, srcp_hbm, dstp_hbm,
               num_out, den_out,
               el_v, er_v, eefull, sidx_all, didx_all, rows4, den4,
               num_acc, den_acc, gsem, ssem, dsem):
    cid = lax.axis_index("c")
    sid = lax.axis_index("s")
    base = sid * SL
    nch = jnp.where(cid == 0, CH0, CH1)
    startrow = jnp.where(cid == 0, sid * CH0, C0TOT + sid * CH1)

    pltpu.sync_copy(elp_hbm, el_v)
    pltpu.sync_copy(erp_hbm, er_v)

    zero = jnp.zeros((L,), jnp.float32)

    for ph, hp_hbm in ((0, h0_hbm), (1, h1_hbm)):
        def zrow(i, _):
            for v in range(DH // L):
                rows4[0, i, pl.ds(v * L, L)] = zero
            den4[0, i, :] = zero
            return 0

        lax.fori_loop(0, G, zrow, 0)

        for k in range(4):
            pltpu.sync_copy(rows4.at[0], num_acc.at[pl.ds(base + k * G, G)])
            if ph == 0:
                pltpu.sync_copy(den4.at[0], den_acc.at[pl.ds(base + k * G, G)])
        rem = SL - 4 * G
        pltpu.sync_copy(rows4.at[0, pl.ds(0, rem)],
                        num_acc.at[pl.ds(base + 4 * G, rem)])
        if ph == 0:
            pltpu.sync_copy(den4.at[0, pl.ds(0, rem)],
                            den_acc.at[pl.ds(base + 4 * G, rem)])
        plsc.subcore_barrier()

        for si in range(2):
            seglen = jnp.clip(nch - si * SEG, 0, SEG)
            segstart = jnp.minimum(startrow + si * SEG, EPG - SEG)
            pltpu.sync_copy(srcp_hbm.at[pl.ds(segstart, SEG)], sidx_all)
            pltpu.sync_copy(dstp_hbm.at[pl.ds(segstart, SEG)], didx_all)

            def eevec(c, _):
                for j in range(G // L):
                    sv = sidx_all[c, pl.ds(j * L, L)]
                    dv = didx_all[c, pl.ds(j * L, L)]
                    s = (plsc.load_gather(el_v, [sv])
                         + plsc.load_gather(er_v, [dv]))
                    e = jnp.where(s >= 0.0, s, 0.2 * s)
                    eefull[pl.ds(c * G + j * L, L)] = jnp.exp(e)
                return 0

            lax.fori_loop(0, seglen, eevec, 0)

            def group(g, _):
                c0 = g * NB
                gd = [pltpu.async_copy(hp_hbm.at[sidx_all.at[c0 + b]],
                                       rows4.at[b], gsem.at[b])
                      for b in range(NB)]
                outd = []
                for b in range(NB):
                    gd[b].wait()
                    loc = (c0 + b) * G

                    def row4(i, _, b=b, loc=loc):
                        for k in range(4):
                            r = i * 4 + k
                            sp = plsc.load_gather(
                                eefull, [jnp.full((L,), loc, jnp.int32) + r])
                            if ph == 0:
                                den4[b, r, :] = sp
                            for v in range(DH // L):
                                rows4[b, r, pl.ds(v * L, L)] = (
                                    rows4[b, r, pl.ds(v * L, L)] * sp)
                        return 0

                    lax.fori_loop(0, G // 4, row4, 0)

                    outd.append(pltpu.async_copy(
                        rows4.at[b], num_acc.at[didx_all.at[c0 + b]],
                        ssem.at[b], add=True))
                    if ph == 0:
                        outd.append(pltpu.async_copy(
                            den4.at[b], den_acc.at[didx_all.at[c0 + b]],
                            dsem.at[b], add=True))
                for d in outd:
                    d.wait()
                return 0

            lax.fori_loop(0, seglen // NB, group, 0)
        plsc.subcore_barrier()

        for k in range(4):
            pltpu.sync_copy(num_acc.at[pl.ds(base + k * G, G)],
                            num_out.at[cid, ph, pl.ds(base + k * G, G)])
            if ph == 0:
                pltpu.sync_copy(den_acc.at[pl.ds(base + k * G, G)],
                                den_out.at[cid, pl.ds(base + k * G, G)])
        pltpu.sync_copy(num_acc.at[pl.ds(base + 4 * G, rem)],
                        num_out.at[cid, ph, pl.ds(base + 4 * G, rem)])
        if ph == 0:
            pltpu.sync_copy(den_acc.at[pl.ds(base + 4 * G, rem)],
                            den_out.at[cid, pl.ds(base + 4 * G, rem)])
        if ph == 0:
            plsc.subcore_barrier()


_edge_kernel = functools.partial(
    pl.kernel,
    out_type=(
        jax.ShapeDtypeStruct((NC, 2, NP, DH), jnp.float32),
        jax.ShapeDtypeStruct((NC, NP, L), jnp.float32),
    ),
    mesh=plsc.VectorSubcoreMesh(core_axis_name="c", subcore_axis_name="s"),
    compiler_params=pltpu.CompilerParams(needs_layout_passes=False,
                                         use_tc_tiling_on_sc=False),
    scratch_types=[
        pltpu.VMEM((NP,), jnp.float32),
        pltpu.VMEM((NP,), jnp.float32),
        pltpu.VMEM((SEG * G,), jnp.float32),
        pltpu.VMEM((SEG, G), jnp.int32),
        pltpu.VMEM((SEG, G), jnp.int32),
        pltpu.VMEM((NB, G, DH), jnp.float32),
        pltpu.VMEM((NB, G, L), jnp.float32),
        pltpu.VMEM_SHARED((NP, DH), jnp.float32),
        pltpu.VMEM_SHARED((NP, L), jnp.float32),
        pltpu.SemaphoreType.DMA((NB,)),
        pltpu.SemaphoreType.DMA((NB,)),
        pltpu.SemaphoreType.DMA((NB,)),
    ],
)(_edge_body)


def _final_body(num_ref, den_ref, bias_ref, out_ref):
    d = den_ref[0, :N, :] + den_ref[1, :N, :]
    d0 = d[:, 0:1]
    for ph in range(2):
        n = num_ref[0, ph, :N, :] + num_ref[1, ph, :N, :]
        out_ref[:, ph * DH:(ph + 1) * DH] = (
            jnp.where(d0 > 0.0, n / d0, 0.0)
            + bias_ref[:, ph * DH:(ph + 1) * DH])


def _finalize(num, den, bias2):
    return pl.pallas_call(
        _final_body,
        out_shape=jax.ShapeDtypeStruct((N, D), jnp.float32),
    )(num, den, bias2)


def kernel(feats, edge_index, W, attn_l, attn_r, bias):
    src = edge_index[0].astype(jnp.int32)
    dst = edge_index[1].astype(jnp.int32)
    srcp = jnp.concatenate(
        [src, jnp.zeros((EP - E,), jnp.int32)]).reshape(EP // G, G)
    pad_dst = N + jnp.arange(EP - E, dtype=jnp.int32) % (NP - N)
    dstp = jnp.concatenate([dst, pad_dst]).reshape(EP // G, G)
    A = jnp.stack([attn_l, attn_r], axis=1)

    h0, h1, elr = _project(feats, W, A)
    elp = jnp.concatenate([elr[:, 0], jnp.zeros((NP - N,), jnp.float32)])
    erp = jnp.concatenate([elr[:, 1], jnp.zeros((NP - N,), jnp.float32)])

    num, den = _edge_kernel(h0, h1, elp, erp, srcp, dstp)
    return _finalize(num, den, bias.reshape(1, D))


# speedup vs baseline: 1.3643x; 1.1337x over previous
import functools

import jax
import jax.numpy as jnp
from jax import lax
from jax.experimental import pallas as pl
from jax.experimental.pallas import tpu as pltpu
from jax.experimental.pallas import tpu_sc as plsc

N = 10000
D = 128
DH = D // 2
E = 320000

NC = 2
NS = 16
L = 16

NP = 10112
SL = NP // NS
EP = 327680
G = 128
CH0 = 112
CH1 = 48
C0TOT = NS * CH0
SEG = 56
EPG = EP // G


def _proj_body(feats_ref, w_ref, a_ref, h0_ref, h1_ref, elr_ref):
    x = feats_ref[...]
    x = jnp.where(x >= 0.0, x, 0.2 * x)
    h = jnp.dot(x, w_ref[...], preferred_element_type=jnp.float32)
    h0_ref[...] = h[:, :DH]
    h1_ref[...] = h[:, DH:]
    elr_ref[...] = jnp.dot(h, a_ref[...], preferred_element_type=jnp.float32)


def _project(feats, W, A):
    return pl.pallas_call(
        _proj_body,
        out_shape=(
            jax.ShapeDtypeStruct((N, DH), jnp.float32),
            jax.ShapeDtypeStruct((N, DH), jnp.float32),
            jax.ShapeDtypeStruct((N, 2), jnp.float32),
        ),
    )(feats, W, A)


NB = 2


def _edge_body(h0_hbm, h1_hbm, elp_hbm, erp_hbm, srcp_hbm, dstp_hbm,
               num_out, den_out,
               el_v, er_v, eefull, sidx_all, didx_all, rows4, den4,
               num_acc, den_acc, gsem, ssem, dsem):
    cid = lax.axis_index("c")
    sid = lax.axis_index("s")
    base = sid * SL
    nch = jnp.where(cid == 0, CH0, CH1)
    startrow = jnp.where(cid == 0, sid * CH0, C0TOT + sid * CH1)

    pltpu.sync_copy(elp_hbm, el_v)
    pltpu.sync_copy(erp_hbm, er_v)

    zero = jnp.zeros((L,), jnp.float32)

    for ph, hp_hbm in ((0, h0_hbm), (1, h1_hbm)):
        def zrow(i, _):
            for v in range(DH // L):
                rows4[0, i, pl.ds(v * L, L)] = zero
            den4[0, i, :] = zero
            return 0

        lax.fori_loop(0, G, zrow, 0)

        for k in range(4):
            pltpu.sync_copy(rows4.at[0], num_acc.at[pl.ds(base + k * G, G)])
            if ph == 0:
                pltpu.sync_copy(den4.at[0], den_acc.at[pl.ds(base + k * G, G)])
        rem = SL - 4 * G
        pltpu.sync_copy(rows4.at[0, pl.ds(0, rem)],
                        num_acc.at[pl.ds(base + 4 * G, rem)])
        if ph == 0:
            pltpu.sync_copy(den4.at[0, pl.ds(0, rem)],
                            den_acc.at[pl.ds(base + 4 * G, rem)])
        plsc.subcore_barrier()

        for si in range(2):
            seglen = jnp.clip(nch - si * SEG, 0, SEG)
            segstart = jnp.minimum(startrow + si * SEG, EPG - SEG)
            pltpu.sync_copy(srcp_hbm.at[pl.ds(segstart, SEG)], sidx_all)
            pltpu.sync_copy(dstp_hbm.at[pl.ds(segstart, SEG)], didx_all)

            def eevec(c, _):
                for j in range(G // L):
                    sv = sidx_all[c, pl.ds(j * L, L)]
                    dv = didx_all[c, pl.ds(j * L, L)]
                    s = (plsc.load_gather(el_v, [sv])
                         + plsc.load_gather(er_v, [dv]))
                    e = jnp.where(s >= 0.0, s, 0.2 * s)
                    eefull[pl.ds(c * G + j * L, L)] = jnp.exp(e)
                return 0

            lax.fori_loop(0, seglen, eevec, 0)

            def group(g, _):
                c0 = g * NB
                gd = [pltpu.async_copy(hp_hbm.at[sidx_all.at[c0 + b]],
                                       rows4.at[b], gsem.at[b])
                      for b in range(NB)]
                outd = []
                for b in range(NB):
                    gd[b].wait()
                    loc = (c0 + b) * G

                    def row4(i, _, b=b, loc=loc):
                        for k in range(4):
                            r = i * 4 + k
                            sp = plsc.load_gather(
                                eefull, [jnp.full((L,), loc, jnp.int32) + r])
                            if ph == 0:
                                den4[b, r, :] = sp
                            for v in range(DH // L):
                                rows4[b, r, pl.ds(v * L, L)] = (
                                    rows4[b, r, pl.ds(v * L, L)] * sp)
                        return 0

                    lax.fori_loop(0, G // 4, row4, 0)

                    outd.append(pltpu.async_copy(
                        rows4.at[b], num_acc.at[didx_all.at[c0 + b]],
                        ssem.at[b], add=True))
                    if ph == 0:
                        outd.append(pltpu.async_copy(
                            den4.at[b], den_acc.at[didx_all.at[c0 + b]],
                            dsem.at[b], add=True))
                for d in outd:
                    d.wait()
                return 0

            lax.fori_loop(0, seglen // NB, group, 0)
        plsc.subcore_barrier()

        for k in range(4):
            pltpu.sync_copy(num_acc.at[pl.ds(base + k * G, G)],
                            num_out.at[cid, ph, pl.ds(base + k * G, G)])
            if ph == 0:
                pltpu.sync_copy(den_acc.at[pl.ds(base + k * G, G)],
                                den_out.at[cid, pl.ds(base + k * G, G)])
        pltpu.sync_copy(num_acc.at[pl.ds(base + 4 * G, rem)],
                        num_out.at[cid, ph, pl.ds(base + 4 * G, rem)])
        if ph == 0:
            pltpu.sync_copy(den_acc.at[pl.ds(base + 4 * G, rem)],
                            den_out.at[cid, pl.ds(base + 4 * G, rem)])
        if ph == 0:
            plsc.subcore_barrier()


_edge_kernel = functools.partial(
    pl.kernel,
    out_type=(
        jax.ShapeDtypeStruct((NC, 2, NP, DH), jnp.float32),
        jax.ShapeDtypeStruct((NC, NP, L), jnp.float32),
    ),
    mesh=plsc.VectorSubcoreMesh(core_axis_name="c", subcore_axis_name="s"),
    compiler_params=pltpu.CompilerParams(needs_layout_passes=False,
                                         use_tc_tiling_on_sc=False),
    scratch_types=[
        pltpu.VMEM((NP,), jnp.float32),
        pltpu.VMEM((NP,), jnp.float32),
        pltpu.VMEM((SEG * G,), jnp.float32),
        pltpu.VMEM((SEG, G), jnp.int32),
        pltpu.VMEM((SEG, G), jnp.int32),
        pltpu.VMEM((NB, G, DH), jnp.float32),
        pltpu.VMEM((NB, G, L), jnp.float32),
        pltpu.VMEM_SHARED((NP, DH), jnp.float32),
        pltpu.VMEM_SHARED((NP, L), jnp.float32),
        pltpu.SemaphoreType.DMA((NB,)),
        pltpu.SemaphoreType.DMA((NB,)),
        pltpu.SemaphoreType.DMA((NB,)),
    ],
)(_edge_body)


def _final_body(num_ref, den_ref, bias_ref, out_ref):
    d = den_ref[0, :N, :] + den_ref[1, :N, :]
    d0 = d[:, 0:1]
    for ph in range(2):
        n = num_ref[0, ph, :N, :] + num_ref[1, ph, :N, :]
        out_ref[:, ph * DH:(ph + 1) * DH] = (
            jnp.where(d0 > 0.0, n / d0, 0.0)
            + bias_ref[:, ph * DH:(ph + 1) * DH])


def _finalize(num, den, bias2):
    return pl.pallas_call(
        _final_body,
        out_shape=jax.ShapeDtypeStruct((N, D), jnp.float32),
    )(num, den, bias2)


def kernel(feats, edge_index, W, attn_l, attn_r, bias):
    src = edge_index[0].astype(jnp.int32)
    dst = edge_index[1].astype(jnp.int32)
    srcp = jnp.concatenate(
        [src, jnp.zeros((EP - E,), jnp.int32)]).reshape(EP // G, G)
    pad_dst = N + jnp.arange(EP - E, dtype=jnp.int32) % (NP - N)
    dstp = jnp.concatenate([dst, pad_dst]).reshape(EP // G, G)
    A = jnp.stack([attn_l, attn_r], axis=1)

    h0, h1, elr = _project(feats, W, A)
    elp = jnp.concatenate([elr[:, 0], jnp.zeros((NP - N,), jnp.float32)])
    erp = jnp.concatenate([elr[:, 1], jnp.zeros((NP - N,), jnp.float32)])

    num, den = _edge_kernel(h0, h1, elp, erp, srcp, dstp)
    return _finalize(num, den, bias.reshape(1, D))


# speedup vs baseline: 1.3876x; 1.0171x over previous
import functools

import jax
import jax.numpy as jnp
from jax import lax
from jax.experimental import pallas as pl
from jax.experimental.pallas import tpu as pltpu
from jax.experimental.pallas import tpu_sc as plsc

N = 10000
D = 128
DH = D // 2
E = 320000

NC = 2
NS = 16
L = 16

NP = 10112
SL = NP // NS
EP = 327680
G = 128
CH0 = 116
CH1 = 44
C0TOT = NS * CH0
SEG = 58
EPG = EP // G


def _proj_body(feats_ref, w_ref, a_ref, h0_ref, h1_ref, elr_ref):
    x = feats_ref[...]
    x = jnp.where(x >= 0.0, x, 0.2 * x)
    h = jnp.dot(x, w_ref[...], preferred_element_type=jnp.float32)
    h0_ref[...] = h[:, :DH]
    h1_ref[...] = h[:, DH:]
    elr_ref[...] = jnp.dot(h, a_ref[...], preferred_element_type=jnp.float32)


def _project(feats, W, A):
    return pl.pallas_call(
        _proj_body,
        out_shape=(
            jax.ShapeDtypeStruct((N, DH), jnp.float32),
            jax.ShapeDtypeStruct((N, DH), jnp.float32),
            jax.ShapeDtypeStruct((N, 2), jnp.float32),
        ),
    )(feats, W, A)


NB = 2


def _edge_body(h0_hbm, h1_hbm, elp_hbm, erp_hbm, srcp_hbm, dstp_hbm,
               num_out, den_out,
               el_v, er_v, eefull, sidx_all, didx_all, rows4, den4,
               num_acc, den_acc, gsem, ssem, dsem):
    cid = lax.axis_index("c")
    sid = lax.axis_index("s")
    base = sid * SL
    nch = jnp.where(cid == 0, CH0, CH1)
    startrow = jnp.where(cid == 0, sid * CH0, C0TOT + sid * CH1)

    pltpu.sync_copy(elp_hbm, el_v)
    pltpu.sync_copy(erp_hbm, er_v)

    zero = jnp.zeros((L,), jnp.float32)

    for ph, hp_hbm in ((0, h0_hbm), (1, h1_hbm)):
        def zrow(i, _):
            for v in range(DH // L):
                rows4[0, i, pl.ds(v * L, L)] = zero
            den4[0, i, :] = zero
            return 0

        lax.fori_loop(0, G, zrow, 0)

        for k in range(4):
            pltpu.sync_copy(rows4.at[0], num_acc.at[pl.ds(base + k * G, G)])
            if ph == 0:
                pltpu.sync_copy(den4.at[0], den_acc.at[pl.ds(base + k * G, G)])
        rem = SL - 4 * G
        pltpu.sync_copy(rows4.at[0, pl.ds(0, rem)],
                        num_acc.at[pl.ds(base + 4 * G, rem)])
        if ph == 0:
            pltpu.sync_copy(den4.at[0, pl.ds(0, rem)],
                            den_acc.at[pl.ds(base + 4 * G, rem)])
        plsc.subcore_barrier()

        for si in range(2):
            seglen = jnp.clip(nch - si * SEG, 0, SEG)
            segstart = jnp.minimum(startrow + si * SEG, EPG - SEG)
            pltpu.sync_copy(srcp_hbm.at[pl.ds(segstart, SEG)], sidx_all)
            pltpu.sync_copy(dstp_hbm.at[pl.ds(segstart, SEG)], didx_all)

            def eevec(c, _):
                for j in range(G // L):
                    sv = sidx_all[c, pl.ds(j * L, L)]
                    dv = didx_all[c, pl.ds(j * L, L)]
                    s = (plsc.load_gather(el_v, [sv])
                         + plsc.load_gather(er_v, [dv]))
                    e = jnp.where(s >= 0.0, s, 0.2 * s)
                    eefull[pl.ds(c * G + j * L, L)] = jnp.exp(e)
                return 0

            lax.fori_loop(0, seglen, eevec, 0)

            def group(g, _):
                c0 = g * NB
                gd = [pltpu.async_copy(hp_hbm.at[sidx_all.at[c0 + b]],
                                       rows4.at[b], gsem.at[b])
                      for b in range(NB)]
                outd = []
                for b in range(NB):
                    gd[b].wait()
                    loc = (c0 + b) * G

                    def row4(i, _, b=b, loc=loc):
                        for k in range(4):
                            r = i * 4 + k
                            sp = plsc.load_gather(
                                eefull, [jnp.full((L,), loc, jnp.int32) + r])
                            if ph == 0:
                                den4[b, r, :] = sp
                            for v in range(DH // L):
                                rows4[b, r, pl.ds(v * L, L)] = (
                                    rows4[b, r, pl.ds(v * L, L)] * sp)
                        return 0

                    lax.fori_loop(0, G // 4, row4, 0)

                    outd.append(pltpu.async_copy(
                        rows4.at[b], num_acc.at[didx_all.at[c0 + b]],
                        ssem.at[b], add=True))
                    if ph == 0:
                        outd.append(pltpu.async_copy(
                            den4.at[b], den_acc.at[didx_all.at[c0 + b]],
                            dsem.at[b], add=True))
                for d in outd:
                    d.wait()
                return 0

            lax.fori_loop(0, seglen // NB, group, 0)
        plsc.subcore_barrier()

        for k in range(4):
            pltpu.sync_copy(num_acc.at[pl.ds(base + k * G, G)],
                            num_out.at[cid, ph, pl.ds(base + k * G, G)])
            if ph == 0:
                pltpu.sync_copy(den_acc.at[pl.ds(base + k * G, G)],
                                den_out.at[cid, pl.ds(base + k * G, G)])
        pltpu.sync_copy(num_acc.at[pl.ds(base + 4 * G, rem)],
                        num_out.at[cid, ph, pl.ds(base + 4 * G, rem)])
        if ph == 0:
            pltpu.sync_copy(den_acc.at[pl.ds(base + 4 * G, rem)],
                            den_out.at[cid, pl.ds(base + 4 * G, rem)])
        if ph == 0:
            plsc.subcore_barrier()


_edge_kernel = functools.partial(
    pl.kernel,
    out_type=(
        jax.ShapeDtypeStruct((NC, 2, NP, DH), jnp.float32),
        jax.ShapeDtypeStruct((NC, NP, L), jnp.float32),
    ),
    mesh=plsc.VectorSubcoreMesh(core_axis_name="c", subcore_axis_name="s"),
    compiler_params=pltpu.CompilerParams(needs_layout_passes=False,
                                         use_tc_tiling_on_sc=False),
    scratch_types=[
        pltpu.VMEM((NP,), jnp.float32),
        pltpu.VMEM((NP,), jnp.float32),
        pltpu.VMEM((SEG * G,), jnp.float32),
        pltpu.VMEM((SEG, G), jnp.int32),
        pltpu.VMEM((SEG, G), jnp.int32),
        pltpu.VMEM((NB, G, DH), jnp.float32),
        pltpu.VMEM((NB, G, L), jnp.float32),
        pltpu.VMEM_SHARED((NP, DH), jnp.float32),
        pltpu.VMEM_SHARED((NP, L), jnp.float32),
        pltpu.SemaphoreType.DMA((NB,)),
        pltpu.SemaphoreType.DMA((NB,)),
        pltpu.SemaphoreType.DMA((NB,)),
    ],
)(_edge_body)


def _final_body(num_ref, den_ref, bias_ref, out_ref):
    d = den_ref[0, :N, :] + den_ref[1, :N, :]
    d0 = d[:, 0:1]
    for ph in range(2):
        n = num_ref[0, ph, :N, :] + num_ref[1, ph, :N, :]
        out_ref[:, ph * DH:(ph + 1) * DH] = (
            jnp.where(d0 > 0.0, n / d0, 0.0)
            + bias_ref[:, ph * DH:(ph + 1) * DH])


def _finalize(num, den, bias2):
    return pl.pallas_call(
        _final_body,
        out_shape=jax.ShapeDtypeStruct((N, D), jnp.float32),
    )(num, den, bias2)


def kernel(feats, edge_index, W, attn_l, attn_r, bias):
    src = edge_index[0].astype(jnp.int32)
    dst = edge_index[1].astype(jnp.int32)
    srcp = jnp.concatenate(
        [src, jnp.zeros((EP - E,), jnp.int32)]).reshape(EP // G, G)
    pad_dst = N + jnp.arange(EP - E, dtype=jnp.int32) % (NP - N)
    dstp = jnp.concatenate([dst, pad_dst]).reshape(EP // G, G)
    A = jnp.stack([attn_l, attn_r], axis=1)

    h0, h1, elr = _project(feats, W, A)
    elp = jnp.concatenate([elr[:, 0], jnp.zeros((NP - N,), jnp.float32)])
    erp = jnp.concatenate([elr[:, 1], jnp.zeros((NP - N,), jnp.float32)])

    num, den = _edge_kernel(h0, h1, elp, erp, srcp, dstp)
    return _finalize(num, den, bias.reshape(1, D))


# speedup vs baseline: 1.3879x; 1.0002x over previous
import functools

import jax
import jax.numpy as jnp
from jax import lax
from jax.experimental import pallas as pl
from jax.experimental.pallas import tpu as pltpu
from jax.experimental.pallas import tpu_sc as plsc

N = 10000
D = 128
DH = D // 2
E = 320000

NC = 2
NS = 16
L = 16

NP = 10112
SL = NP // NS
EP = 327680
G = 128
CH0 = 114
CH1 = 46
C0TOT = NS * CH0
SEG = 58
EPG = EP // G


def _proj_body(feats_ref, w_ref, a_ref, h0_ref, h1_ref, elr_ref):
    x = feats_ref[...]
    x = jnp.where(x >= 0.0, x, 0.2 * x)
    h = jnp.dot(x, w_ref[...], preferred_element_type=jnp.float32)
    h0_ref[...] = h[:, :DH]
    h1_ref[...] = h[:, DH:]
    elr_ref[...] = jnp.dot(h, a_ref[...], preferred_element_type=jnp.float32)


def _project(feats, W, A):
    return pl.pallas_call(
        _proj_body,
        out_shape=(
            jax.ShapeDtypeStruct((N, DH), jnp.float32),
            jax.ShapeDtypeStruct((N, DH), jnp.float32),
            jax.ShapeDtypeStruct((N, 2), jnp.float32),
        ),
    )(feats, W, A)


NB = 2


def _edge_body(h0_hbm, h1_hbm, elp_hbm, erp_hbm, srcp_hbm, dstp_hbm,
               num_out, den_out,
               el_v, er_v, eefull, sidx_all, didx_all, rows4, den4,
               num_acc, den_acc, gsem, ssem, dsem):
    cid = lax.axis_index("c")
    sid = lax.axis_index("s")
    base = sid * SL
    nch = jnp.where(cid == 0, CH0, CH1)
    startrow = jnp.where(cid == 0, sid * CH0, C0TOT + sid * CH1)

    pltpu.sync_copy(elp_hbm, el_v)
    pltpu.sync_copy(erp_hbm, er_v)

    zero = jnp.zeros((L,), jnp.float32)

    for ph, hp_hbm in ((0, h0_hbm), (1, h1_hbm)):
        def zrow(i, _):
            for v in range(DH // L):
                rows4[0, i, pl.ds(v * L, L)] = zero
            den4[0, i, :] = zero
            return 0

        lax.fori_loop(0, G, zrow, 0)

        for k in range(4):
            pltpu.sync_copy(rows4.at[0], num_acc.at[pl.ds(base + k * G, G)])
            if ph == 0:
                pltpu.sync_copy(den4.at[0], den_acc.at[pl.ds(base + k * G, G)])
        rem = SL - 4 * G
        pltpu.sync_copy(rows4.at[0, pl.ds(0, rem)],
                        num_acc.at[pl.ds(base + 4 * G, rem)])
        if ph == 0:
            pltpu.sync_copy(den4.at[0, pl.ds(0, rem)],
                            den_acc.at[pl.ds(base + 4 * G, rem)])
        plsc.subcore_barrier()

        for si in range(2):
            seglen = jnp.clip(nch - si * SEG, 0, SEG)
            segstart = jnp.minimum(startrow + si * SEG, EPG - SEG)
            pltpu.sync_copy(srcp_hbm.at[pl.ds(segstart, SEG)], sidx_all)
            pltpu.sync_copy(dstp_hbm.at[pl.ds(segstart, SEG)], didx_all)

            def eevec(c, _):
                for j in range(G // L):
                    sv = sidx_all[c, pl.ds(j * L, L)]
                    dv = didx_all[c, pl.ds(j * L, L)]
                    s = (plsc.load_gather(el_v, [sv])
                         + plsc.load_gather(er_v, [dv]))
                    e = jnp.where(s >= 0.0, s, 0.2 * s)
                    eefull[pl.ds(c * G + j * L, L)] = jnp.exp(e)
                return 0

            lax.fori_loop(0, seglen, eevec, 0)

            def group(g, _):
                c0 = g * NB
                gd = [pltpu.async_copy(hp_hbm.at[sidx_all.at[c0 + b]],
                                       rows4.at[b], gsem.at[b])
                      for b in range(NB)]
                outd = []
                for b in range(NB):
                    gd[b].wait()
                    loc = (c0 + b) * G

                    def row4(i, _, b=b, loc=loc):
                        for k in range(4):
                            r = i * 4 + k
                            sp = plsc.load_gather(
                                eefull, [jnp.full((L,), loc, jnp.int32) + r])
                            if ph == 0:
                                den4[b, r, :] = sp
                            for v in range(DH // L):
                                rows4[b, r, pl.ds(v * L, L)] = (
                                    rows4[b, r, pl.ds(v * L, L)] * sp)
                        return 0

                    lax.fori_loop(0, G // 4, row4, 0)

                    outd.append(pltpu.async_copy(
                        rows4.at[b], num_acc.at[didx_all.at[c0 + b]],
                        ssem.at[b], add=True))
                    if ph == 0:
                        outd.append(pltpu.async_copy(
                            den4.at[b], den_acc.at[didx_all.at[c0 + b]],
                            dsem.at[b], add=True))
                for d in outd:
                    d.wait()
                return 0

            lax.fori_loop(0, seglen // NB, group, 0)
        plsc.subcore_barrier()

        for k in range(4):
            pltpu.sync_copy(num_acc.at[pl.ds(base + k * G, G)],
                            num_out.at[cid, ph, pl.ds(base + k * G, G)])
            if ph == 0:
                pltpu.sync_copy(den_acc.at[pl.ds(base + k * G, G)],
                                den_out.at[cid, pl.ds(base + k * G, G)])
        pltpu.sync_copy(num_acc.at[pl.ds(base + 4 * G, rem)],
                        num_out.at[cid, ph, pl.ds(base + 4 * G, rem)])
        if ph == 0:
            pltpu.sync_copy(den_acc.at[pl.ds(base + 4 * G, rem)],
                            den_out.at[cid, pl.ds(base + 4 * G, rem)])
        if ph == 0:
            plsc.subcore_barrier()


_edge_kernel = functools.partial(
    pl.kernel,
    out_type=(
        jax.ShapeDtypeStruct((NC, 2, NP, DH), jnp.float32),
        jax.ShapeDtypeStruct((NC, NP, L), jnp.float32),
    ),
    mesh=plsc.VectorSubcoreMesh(core_axis_name="c", subcore_axis_name="s"),
    compiler_params=pltpu.CompilerParams(needs_layout_passes=False,
                                         use_tc_tiling_on_sc=False),
    scratch_types=[
        pltpu.VMEM((NP,), jnp.float32),
        pltpu.VMEM((NP,), jnp.float32),
        pltpu.VMEM((SEG * G,), jnp.float32),
        pltpu.VMEM((SEG, G), jnp.int32),
        pltpu.VMEM((SEG, G), jnp.int32),
        pltpu.VMEM((NB, G, DH), jnp.float32),
        pltpu.VMEM((NB, G, L), jnp.float32),
        pltpu.VMEM_SHARED((NP, DH), jnp.float32),
        pltpu.VMEM_SHARED((NP, L), jnp.float32),
        pltpu.SemaphoreType.DMA((NB,)),
        pltpu.SemaphoreType.DMA((NB,)),
        pltpu.SemaphoreType.DMA((NB,)),
    ],
)(_edge_body)


def _final_body(num_ref, den_ref, bias_ref, out_ref):
    d = den_ref[0, :N, :] + den_ref[1, :N, :]
    d0 = d[:, 0:1]
    for ph in range(2):
        n = num_ref[0, ph, :N, :] + num_ref[1, ph, :N, :]
        out_ref[:, ph * DH:(ph + 1) * DH] = (
            jnp.where(d0 > 0.0, n / d0, 0.0)
            + bias_ref[:, ph * DH:(ph + 1) * DH])


def _finalize(num, den, bias2):
    return pl.pallas_call(
        _final_body,
        out_shape=jax.ShapeDtypeStruct((N, D), jnp.float32),
    )(num, den, bias2)


def kernel(feats, edge_index, W, attn_l, attn_r, bias):
    src = edge_index[0].astype(jnp.int32)
    dst = edge_index[1].astype(jnp.int32)
    srcp = jnp.concatenate(
        [src, jnp.zeros((EP - E,), jnp.int32)]).reshape(EP // G, G)
    pad_dst = N + jnp.arange(EP - E, dtype=jnp.int32) % (NP - N)
    dstp = jnp.concatenate([dst, pad_dst]).reshape(EP // G, G)
    A = jnp.stack([attn_l, attn_r], axis=1)

    h0, h1, elr = _project(feats, W, A)
    elp = jnp.concatenate([elr[:, 0], jnp.zeros((NP - N,), jnp.float32)])
    erp = jnp.concatenate([elr[:, 1], jnp.zeros((NP - N,), jnp.float32)])

    num, den = _edge_kernel(h0, h1, elp, erp, srcp, dstp)
    return _finalize(num, den, bias.reshape(1, D))
